# baseline (speedup 1.0000x reference)
import jax
import jax.numpy as jnp
from jax import lax
from jax.experimental import pallas as pl
from jax.experimental.pallas import tpu as pltpu
from jax.experimental.pallas import tpu_sc as plsc

N = 20000
K = 100
IOU_THR = 0.5
NSUB = 16
NPAD = 20480
SHARD = NPAD // NSUB
CHUNKS = SHARD // 16
NEG = -1.0e30
BIGF = 1.0e30
KPAD = 128


def _nms_body(x1_h, y1_h, x2_h, y2_h, s_h, osc_h, obox_h, oidx_h,
              x1_v, y1_v, x2_v, y2_v, ar_v, s_v,
              cand_v, allc_v, osc_v, obox_v, oidx_v, shared_a, shared_b, sem):
    cid = lax.axis_index("c")
    wid = lax.axis_index("s")
    base = wid * SHARD

    iota_i = lax.iota(jnp.int32, 16)
    iota_f = iota_i.astype(jnp.float32)
    zeros16 = jnp.zeros((16,), jnp.float32)

    cps = [pltpu.make_async_copy(h.at[pl.ds(base, SHARD)], v, sem)
           for h, v in ((x1_h, x1_v), (y1_h, y1_v), (x2_h, x2_v),
                        (y2_h, y2_v), (s_h, s_v))]
    for cp in cps:
        cp.start()
    for cp in cps:
        cp.wait()

    for c in range(CHUNKS):
        sl = pl.ds(c * 16, 16)
        ar_v[sl] = (x2_v[sl] - x1_v[sl] + 1.0) * (y2_v[sl] - y1_v[sl] + 1.0)

    for j in range(KPAD // 16):
        sl = pl.ds(j * 16, 16)
        osc_v[sl] = zeros16
        oidx_v[sl] = jnp.full((16,), -1, jnp.int32)
    for j in range(4 * KPAD // 16):
        obox_v[pl.ds(j * 16, 16)] = zeros16

    basef = jnp.float32(base)

    def one_round(k, carry, board):
        kx1, ky1, kx2, ky2, kar = carry
        kx1v = jnp.full((16,), kx1)
        ky1v = jnp.full((16,), ky1)
        kx2v = jnp.full((16,), kx2)
        ky2v = jnp.full((16,), ky2)
        karv = jnp.full((16,), kar)

        def chunk_body(c, cc):
            m_best, i_best = cc
            sl = pl.ds(c * 16, 16)
            xmn = jnp.maximum(x1_v[sl], kx1v)
            ymn = jnp.maximum(y1_v[sl], ky1v)
            xmx = jnp.minimum(x2_v[sl], kx2v)
            ymx = jnp.minimum(y2_v[sl], ky2v)
            inter = jnp.maximum(xmx - xmn, 0.0) * jnp.maximum(ymx - ymn, 0.0)
            denom = (karv + ar_v[sl]) - inter
            sup = inter > IOU_THR * denom
            sc = jnp.where(sup, NEG, s_v[sl])
            s_v[sl] = sc
            upd = sc > m_best
            m_best = jnp.where(upd, sc, m_best)
            i_best = jnp.where(upd,
                               iota_f + (basef + c.astype(jnp.float32) * 16.0),
                               i_best)
            return (m_best, i_best)

        m_best, i_best = lax.fori_loop(
            0, CHUNKS, chunk_body,
            (jnp.full((16,), -3.0e38, jnp.float32),
             jnp.full((16,), BIGF, jnp.float32)))

        m_loc = jnp.max(m_best)
        i_loc = jnp.min(jnp.where(m_best == m_loc, i_best, BIGF))

        off = (i_loc - basef).astype(jnp.int32)
        blk = jnp.bitwise_and(off, -16)
        lane = jnp.bitwise_and(off, 15)
        lsel = iota_i == lane
        bs = pl.ds(blk, 16)
        cx1 = jnp.max(jnp.where(lsel, x1_v[bs], -BIGF))
        cy1 = jnp.max(jnp.where(lsel, y1_v[bs], -BIGF))
        cx2 = jnp.max(jnp.where(lsel, x2_v[bs], -BIGF))
        cy2 = jnp.max(jnp.where(lsel, y2_v[bs], -BIGF))

        cand = jnp.where(iota_i == 0, jnp.full((16,), m_loc),
               jnp.where(iota_i == 1, jnp.full((16,), i_loc),
               jnp.where(iota_i == 2, jnp.full((16,), cx1),
               jnp.where(iota_i == 3, jnp.full((16,), cy1),
               jnp.where(iota_i == 4, jnp.full((16,), cx2),
               jnp.where(iota_i == 5, jnp.full((16,), cy2), zeros16))))))
        cand_v[...] = cand
        pltpu.sync_copy(cand_v, board.at[pl.ds(wid * 16, 16)])
        plsc.subcore_barrier()
        pltpu.sync_copy(board, allc_v)
        plsc.subcore_barrier()

        m_b = jnp.float32(-3.0e38)
        i_b = jnp.float32(BIGF)
        x1_b = jnp.float32(0.0)
        y1_b = jnp.float32(0.0)
        x2_b = jnp.float32(0.0)
        y2_b = jnp.float32(0.0)
        for r in range(NSUB):
            row = allc_v[pl.ds(r * 16, 16)]
            m_r = row[0]
            i_r = row[1]
            better = (m_r > m_b) | ((m_r == m_b) & (i_r < i_b))
            m_b = jnp.where(better, m_r, m_b)
            i_b = jnp.where(better, i_r, i_b)
            x1_b = jnp.where(better, row[2], x1_b)
            y1_b = jnp.where(better, row[3], y1_b)
            x2_b = jnp.where(better, row[4], x2_b)
            y2_b = jnp.where(better, row[5], y2_b)

        valid = m_b > -1.0e29
        validv = jnp.full((16,), valid)

        own = (i_b >= basef) & (i_b < basef + SHARD)
        goff = jnp.clip(i_b - basef, 0.0, SHARD - 1.0).astype(jnp.int32)
        gblk = jnp.bitwise_and(goff, -16)
        glane = jnp.bitwise_and(goff, 15)
        gs = pl.ds(gblk, 16)
        kill = jnp.full((16,), own) & (iota_i == glane)
        s_v[gs] = jnp.where(kill, NEG, s_v[gs])

        k_blk = jnp.bitwise_and(k, -16)
        k_lane = jnp.bitwise_and(k, 15)
        ksel = (iota_i == k_lane) & validv
        ks = pl.ds(k_blk, 16)
        osc_v[ks] = jnp.where(ksel, jnp.full((16,), m_b), osc_v[ks])
        oidx_v[ks] = jnp.where(ksel, jnp.full((16,), i_b.astype(jnp.int32)),
                               oidx_v[ks])
        b0 = 4 * k
        b_blk = jnp.bitwise_and(b0, -16)
        p = jnp.bitwise_and(b0, 15)
        bsl = pl.ds(b_blk, 16)
        brow = obox_v[bsl]
        bv = jnp.where((iota_i == p) & validv, jnp.full((16,), x1_b),
             jnp.where((iota_i == p + 1) & validv, jnp.full((16,), y1_b),
             jnp.where((iota_i == p + 2) & validv, jnp.full((16,), x2_b),
             jnp.where((iota_i == p + 3) & validv, jnp.full((16,), y2_b),
                       brow))))
        obox_v[bsl] = bv

        karn = (x2_b - x1_b + 1.0) * (y2_b - y1_b + 1.0)
        nkx1 = jnp.where(valid, x1_b, -1.0e6)
        nky1 = jnp.where(valid, y1_b, -1.0e6)
        nkx2 = jnp.where(valid, x2_b, -1.0e6)
        nky2 = jnp.where(valid, y2_b, -1.0e6)
        nkar = jnp.where(valid, karn, 1.0)
        return (nkx1, nky1, nkx2, nky2, nkar)

    init = (jnp.float32(-1.0e6), jnp.float32(-1.0e6), jnp.float32(-1.0e6),
            jnp.float32(-1.0e6), jnp.float32(1.0))
    lax.fori_loop(0, K, lambda k, c: one_round(k, c, shared_a), init,
                  unroll=False)

    @pl.when((cid == 0) & (wid == 0))
    def _():
        pltpu.sync_copy(osc_v, osc_h)
        pltpu.sync_copy(obox_v, obox_h)
        pltpu.sync_copy(oidx_v, oidx_h)


@jax.jit
def kernel(boxes, scores):
    f32 = jnp.float32
    pad = NPAD - N
    x1 = jnp.concatenate([boxes[:, 0], jnp.full((pad,), -1.0e6, f32)])
    y1 = jnp.concatenate([boxes[:, 1], jnp.full((pad,), -1.0e6, f32)])
    x2 = jnp.concatenate([boxes[:, 2], jnp.full((pad,), -1.0e6, f32)])
    y2 = jnp.concatenate([boxes[:, 3], jnp.full((pad,), -1.0e6, f32)])
    s = jnp.concatenate([scores.astype(f32), jnp.full((pad,), NEG, f32)])

    mesh = plsc.VectorSubcoreMesh(core_axis_name="c", subcore_axis_name="s",
                                  num_cores=1, num_subcores=NSUB)
    run = pl.kernel(
        _nms_body,
        out_type=(
            jax.ShapeDtypeStruct((KPAD,), f32),
            jax.ShapeDtypeStruct((4 * KPAD,), f32),
            jax.ShapeDtypeStruct((KPAD,), jnp.int32),
        ),
        mesh=mesh,
        compiler_params=pltpu.CompilerParams(needs_layout_passes=False),
        scratch_types=[
            pltpu.VMEM((SHARD,), f32),
            pltpu.VMEM((SHARD,), f32),
            pltpu.VMEM((SHARD,), f32),
            pltpu.VMEM((SHARD,), f32),
            pltpu.VMEM((SHARD,), f32),
            pltpu.VMEM((SHARD,), f32),
            pltpu.VMEM((16,), f32),
            pltpu.VMEM((NSUB * 16,), f32),
            pltpu.VMEM((KPAD,), f32),
            pltpu.VMEM((4 * KPAD,), f32),
            pltpu.VMEM((KPAD,), jnp.int32),
            pltpu.VMEM_SHARED((NSUB * 16,), f32),
            pltpu.VMEM_SHARED((NSUB * 16,), f32),
            pltpu.SemaphoreType.DMA,
        ],
    )
    osc, obox, oidx = run(x1, y1, x2, y2, s)
    return osc[:K], obox.reshape(KPAD, 4)[:K], oidx[:K]


# speedup vs baseline: 1.7478x; 1.7478x over previous
import jax
import jax.numpy as jnp
from jax import lax
from jax.experimental import pallas as pl
from jax.experimental.pallas import tpu as pltpu
from jax.experimental.pallas import tpu_sc as plsc

N = 20000
K = 100
IOU_THR = 0.5
NSUB = 16
NPAD = 20480
SHARD = NPAD // NSUB
CHUNKS = SHARD // 16
NEG = -1.0e30
BIGF = 1.0e30
KPAD = 128


def _nms_body(x1_h, y1_h, x2_h, y2_h, s_h, osc_h, obox_h, oidx_h,
              x1_v, y1_v, x2_v, y2_v, ar_v, s_v,
              cand_v, allc_v, osc_v, obox_v, oidx_v, board2, sem):
    cid = lax.axis_index("c")
    wid = lax.axis_index("s")
    base = wid * SHARD

    iota_i = lax.iota(jnp.int32, 16)
    iota_f = iota_i.astype(jnp.float32)
    zeros16 = jnp.zeros((16,), jnp.float32)

    cps = [pltpu.make_async_copy(h.at[pl.ds(base, SHARD)], v, sem)
           for h, v in ((x1_h, x1_v), (y1_h, y1_v), (x2_h, x2_v),
                        (y2_h, y2_v), (s_h, s_v))]
    for cp in cps:
        cp.start()
    for cp in cps:
        cp.wait()

    for c in range(CHUNKS):
        sl = pl.ds(c * 16, 16)
        ar_v[sl] = (x2_v[sl] - x1_v[sl] + 1.0) * (y2_v[sl] - y1_v[sl] + 1.0)

    for j in range(KPAD // 16):
        sl = pl.ds(j * 16, 16)
        osc_v[sl] = zeros16
        oidx_v[sl] = jnp.full((16,), -1, jnp.int32)
    for j in range(4 * KPAD // 16):
        obox_v[pl.ds(j * 16, 16)] = zeros16

    basef = jnp.float32(base)

    def one_round(k, carry):
        kx1, ky1, kx2, ky2, kar = carry
        kx1v = jnp.full((16,), kx1)
        ky1v = jnp.full((16,), ky1)
        kx2v = jnp.full((16,), kx2)
        ky2v = jnp.full((16,), ky2)
        karv = jnp.full((16,), kar)

        m_best = jnp.full((16,), -3.0e38, jnp.float32)
        i_best = jnp.full((16,), BIGF, jnp.float32)
        for c in range(CHUNKS):
            sl = pl.ds(c * 16, 16)
            xmn = jnp.maximum(x1_v[sl], kx1v)
            ymn = jnp.maximum(y1_v[sl], ky1v)
            xmx = jnp.minimum(x2_v[sl], kx2v)
            ymx = jnp.minimum(y2_v[sl], ky2v)
            inter = jnp.maximum(xmx - xmn, 0.0) * jnp.maximum(ymx - ymn, 0.0)
            denom = (karv + ar_v[sl]) - inter
            sup = inter > IOU_THR * denom
            sc = jnp.where(sup, NEG, s_v[sl])
            s_v[sl] = sc
            upd = sc > m_best
            m_best = jnp.where(upd, sc, m_best)
            i_best = jnp.where(upd, iota_f + (basef + c * 16.0), i_best)

        m_loc = jnp.max(m_best)
        i_loc = jnp.min(jnp.where(m_best == m_loc, i_best, BIGF))

        off = (i_loc - basef).astype(jnp.int32)
        blk = jnp.bitwise_and(off, -16)
        lane = jnp.bitwise_and(off, 15)
        lsel = iota_i == lane
        bs = pl.ds(blk, 16)
        cx1 = jnp.max(jnp.where(lsel, x1_v[bs], -BIGF))
        cy1 = jnp.max(jnp.where(lsel, y1_v[bs], -BIGF))
        cx2 = jnp.max(jnp.where(lsel, x2_v[bs], -BIGF))
        cy2 = jnp.max(jnp.where(lsel, y2_v[bs], -BIGF))

        cand = jnp.where(iota_i == 0, jnp.full((16,), m_loc),
               jnp.where(iota_i == 1, jnp.full((16,), i_loc),
               jnp.where(iota_i == 2, jnp.full((16,), cx1),
               jnp.where(iota_i == 3, jnp.full((16,), cy1),
               jnp.where(iota_i == 4, jnp.full((16,), cx2),
               jnp.where(iota_i == 5, jnp.full((16,), cy2), zeros16))))))
        boff = jnp.bitwise_and(k, 1) * (NSUB * 16)
        cand_v[...] = cand
        pltpu.sync_copy(cand_v, board2.at[pl.ds(boff + wid * 16, 16)])
        plsc.subcore_barrier()
        pltpu.sync_copy(board2.at[pl.ds(boff, NSUB * 16)], allc_v)

        m_b = jnp.float32(-3.0e38)
        i_b = jnp.float32(BIGF)
        x1_b = jnp.float32(0.0)
        y1_b = jnp.float32(0.0)
        x2_b = jnp.float32(0.0)
        y2_b = jnp.float32(0.0)
        for r in range(NSUB):
            row = allc_v[pl.ds(r * 16, 16)]
            m_r = row[0]
            i_r = row[1]
            better = (m_r > m_b) | ((m_r == m_b) & (i_r < i_b))
            m_b = jnp.where(better, m_r, m_b)
            i_b = jnp.where(better, i_r, i_b)
            x1_b = jnp.where(better, row[2], x1_b)
            y1_b = jnp.where(better, row[3], y1_b)
            x2_b = jnp.where(better, row[4], x2_b)
            y2_b = jnp.where(better, row[5], y2_b)

        valid = m_b > -1.0e29
        validv = jnp.full((16,), valid)

        own = (i_b >= basef) & (i_b < basef + SHARD)
        goff = jnp.clip(i_b - basef, 0.0, SHARD - 1.0).astype(jnp.int32)
        gblk = jnp.bitwise_and(goff, -16)
        glane = jnp.bitwise_and(goff, 15)
        gs = pl.ds(gblk, 16)
        kill = jnp.full((16,), own) & (iota_i == glane)
        s_v[gs] = jnp.where(kill, NEG, s_v[gs])

        k_blk = jnp.bitwise_and(k, -16)
        k_lane = jnp.bitwise_and(k, 15)
        ksel = (iota_i == k_lane) & validv
        ks = pl.ds(k_blk, 16)
        osc_v[ks] = jnp.where(ksel, jnp.full((16,), m_b), osc_v[ks])
        oidx_v[ks] = jnp.where(ksel, jnp.full((16,), i_b.astype(jnp.int32)),
                               oidx_v[ks])
        b0 = 4 * k
        b_blk = jnp.bitwise_and(b0, -16)
        p = jnp.bitwise_and(b0, 15)
        bsl = pl.ds(b_blk, 16)
        brow = obox_v[bsl]
        bv = jnp.where((iota_i == p) & validv, jnp.full((16,), x1_b),
             jnp.where((iota_i == p + 1) & validv, jnp.full((16,), y1_b),
             jnp.where((iota_i == p + 2) & validv, jnp.full((16,), x2_b),
             jnp.where((iota_i == p + 3) & validv, jnp.full((16,), y2_b),
                       brow))))
        obox_v[bsl] = bv

        karn = (x2_b - x1_b + 1.0) * (y2_b - y1_b + 1.0)
        nkx1 = jnp.where(valid, x1_b, -1.0e6)
        nky1 = jnp.where(valid, y1_b, -1.0e6)
        nkx2 = jnp.where(valid, x2_b, -1.0e6)
        nky2 = jnp.where(valid, y2_b, -1.0e6)
        nkar = jnp.where(valid, karn, 1.0)
        return (nkx1, nky1, nkx2, nky2, nkar)

    init = (jnp.float32(-1.0e6), jnp.float32(-1.0e6), jnp.float32(-1.0e6),
            jnp.float32(-1.0e6), jnp.float32(1.0))
    lax.fori_loop(0, K, one_round, init, unroll=False)

    @pl.when((cid == 0) & (wid == 0))
    def _():
        pltpu.sync_copy(osc_v, osc_h)
        pltpu.sync_copy(obox_v, obox_h)
        pltpu.sync_copy(oidx_v, oidx_h)


@jax.jit
def kernel(boxes, scores):
    f32 = jnp.float32
    pad = NPAD - N
    x1 = jnp.concatenate([boxes[:, 0], jnp.full((pad,), -1.0e6, f32)])
    y1 = jnp.concatenate([boxes[:, 1], jnp.full((pad,), -1.0e6, f32)])
    x2 = jnp.concatenate([boxes[:, 2], jnp.full((pad,), -1.0e6, f32)])
    y2 = jnp.concatenate([boxes[:, 3], jnp.full((pad,), -1.0e6, f32)])
    s = jnp.concatenate([scores.astype(f32), jnp.full((pad,), NEG, f32)])

    mesh = plsc.VectorSubcoreMesh(core_axis_name="c", subcore_axis_name="s",
                                  num_cores=1, num_subcores=NSUB)
    run = pl.kernel(
        _nms_body,
        out_type=(
            jax.ShapeDtypeStruct((KPAD,), f32),
            jax.ShapeDtypeStruct((4 * KPAD,), f32),
            jax.ShapeDtypeStruct((KPAD,), jnp.int32),
        ),
        mesh=mesh,
        compiler_params=pltpu.CompilerParams(needs_layout_passes=False),
        scratch_types=[
            pltpu.VMEM((SHARD,), f32),
            pltpu.VMEM((SHARD,), f32),
            pltpu.VMEM((SHARD,), f32),
            pltpu.VMEM((SHARD,), f32),
            pltpu.VMEM((SHARD,), f32),
            pltpu.VMEM((SHARD,), f32),
            pltpu.VMEM((16,), f32),
            pltpu.VMEM((NSUB * 16,), f32),
            pltpu.VMEM((KPAD,), f32),
            pltpu.VMEM((4 * KPAD,), f32),
            pltpu.VMEM((KPAD,), jnp.int32),
            pltpu.VMEM_SHARED((2 * NSUB * 16,), f32),
            pltpu.SemaphoreType.DMA,
        ],
    )
    osc, obox, oidx = run(x1, y1, x2, y2, s)
    return osc[:K], obox.reshape(KPAD, 4)[:K], oidx[:K]


# speedup vs baseline: 1.7683x; 1.0117x over previous
import jax
import jax.numpy as jnp
from jax import lax
from jax.experimental import pallas as pl
from jax.experimental.pallas import tpu as pltpu
from jax.experimental.pallas import tpu_sc as plsc

N = 20000
K = 100
IOU_THR = 0.5
NSUB = 16
NPAD = 20480
SHARD = NPAD // NSUB
CHUNKS = SHARD // 16
NEG = -1.0e30
BIGF = 1.0e30
KPAD = 128


def _nms_body(x1_h, y1_h, x2_h, y2_h, s_h, osc_h, obox_h, oidx_h,
              x1_v, y1_v, x2_v, y2_v, ar_v, s_v,
              cand_v, allc_v, osc_v, obox_v, oidx_v, board2, sem):
    cid = lax.axis_index("c")
    wid = lax.axis_index("s")
    base = wid * SHARD

    iota_i = lax.iota(jnp.int32, 16)
    iota_f = iota_i.astype(jnp.float32)
    zeros16 = jnp.zeros((16,), jnp.float32)

    cps = [pltpu.make_async_copy(h.at[pl.ds(base, SHARD)], v, sem)
           for h, v in ((x1_h, x1_v), (y1_h, y1_v), (x2_h, x2_v),
                        (y2_h, y2_v), (s_h, s_v))]
    for cp in cps:
        cp.start()
    for cp in cps:
        cp.wait()

    for c in range(CHUNKS):
        sl = pl.ds(c * 16, 16)
        ar_v[sl] = (x2_v[sl] - x1_v[sl] + 1.0) * (y2_v[sl] - y1_v[sl] + 1.0)

    for j in range(KPAD // 16):
        sl = pl.ds(j * 16, 16)
        osc_v[sl] = zeros16
        oidx_v[sl] = jnp.full((16,), -1, jnp.int32)
    for j in range(4 * KPAD // 16):
        obox_v[pl.ds(j * 16, 16)] = zeros16

    basef = jnp.float32(base)

    def one_round(k, carry):
        kx1, ky1, kx2, ky2, kar = carry
        kx1v = jnp.full((16,), kx1)
        ky1v = jnp.full((16,), ky1)
        kx2v = jnp.full((16,), kx2)
        ky2v = jnp.full((16,), ky2)
        karv = jnp.full((16,), kar)

        m_best = jnp.full((16,), -3.0e38, jnp.float32)
        i_best = jnp.full((16,), BIGF, jnp.float32)
        for c in range(CHUNKS):
            sl = pl.ds(c * 16, 16)
            xmn = jnp.maximum(x1_v[sl], kx1v)
            ymn = jnp.maximum(y1_v[sl], ky1v)
            xmx = jnp.minimum(x2_v[sl], kx2v)
            ymx = jnp.minimum(y2_v[sl], ky2v)
            inter = jnp.maximum(xmx - xmn, 0.0) * jnp.maximum(ymx - ymn, 0.0)
            denom = (karv + ar_v[sl]) - inter
            sup = inter > IOU_THR * denom
            sc = jnp.where(sup, NEG, s_v[sl])
            s_v[sl] = sc
            upd = sc > m_best
            m_best = jnp.where(upd, sc, m_best)
            i_best = jnp.where(upd, iota_f + (basef + c * 16.0), i_best)

        m_loc = jnp.max(m_best)
        i_loc = jnp.min(jnp.where(m_best == m_loc, i_best, BIGF))

        off = (i_loc - basef).astype(jnp.int32)
        blk = jnp.bitwise_and(off, -16)
        lane = jnp.bitwise_and(off, 15)
        lanev = jnp.full((16, 1), lane, jnp.int32)
        dnums = lax.GatherDimensionNumbers(
            offset_dims=(), collapsed_slice_dims=(0,), start_index_map=(0,))

        def bcast_lane(vec):
            return lax.gather(vec, lanev, dnums, slice_sizes=(1,),
                              mode=lax.GatherScatterMode.PROMISE_IN_BOUNDS)

        bs = pl.ds(blk, 16)
        cx1v = bcast_lane(x1_v[bs])
        cy1v = bcast_lane(y1_v[bs])
        cx2v = bcast_lane(x2_v[bs])
        cy2v = bcast_lane(y2_v[bs])

        cand = jnp.where(iota_i == 0, jnp.full((16,), m_loc),
               jnp.where(iota_i == 1, jnp.full((16,), i_loc),
               jnp.where(iota_i == 2, cx1v,
               jnp.where(iota_i == 3, cy1v,
               jnp.where(iota_i == 4, cx2v,
               jnp.where(iota_i == 5, cy2v, zeros16))))))
        boff = jnp.bitwise_and(k, 1) * (NSUB * 16)
        cand_v[...] = cand
        pltpu.sync_copy(cand_v, board2.at[pl.ds(boff + wid * 16, 16)])
        plsc.subcore_barrier()
        pltpu.sync_copy(board2.at[pl.ds(boff, NSUB * 16)], allc_v)

        m_b = jnp.float32(-3.0e38)
        i_b = jnp.float32(BIGF)
        x1_b = jnp.float32(0.0)
        y1_b = jnp.float32(0.0)
        x2_b = jnp.float32(0.0)
        y2_b = jnp.float32(0.0)
        for r in range(NSUB):
            row = allc_v[pl.ds(r * 16, 16)]
            m_r = row[0]
            i_r = row[1]
            better = (m_r > m_b) | ((m_r == m_b) & (i_r < i_b))
            m_b = jnp.where(better, m_r, m_b)
            i_b = jnp.where(better, i_r, i_b)
            x1_b = jnp.where(better, row[2], x1_b)
            y1_b = jnp.where(better, row[3], y1_b)
            x2_b = jnp.where(better, row[4], x2_b)
            y2_b = jnp.where(better, row[5], y2_b)

        valid = m_b > -1.0e29
        validv = jnp.full((16,), valid)

        own = (i_b >= basef) & (i_b < basef + SHARD)
        goff = jnp.clip(i_b - basef, 0.0, SHARD - 1.0).astype(jnp.int32)
        gblk = jnp.bitwise_and(goff, -16)
        glane = jnp.bitwise_and(goff, 15)
        gs = pl.ds(gblk, 16)
        kill = jnp.full((16,), own) & (iota_i == glane)
        s_v[gs] = jnp.where(kill, NEG, s_v[gs])

        k_blk = jnp.bitwise_and(k, -16)
        k_lane = jnp.bitwise_and(k, 15)
        ksel = (iota_i == k_lane) & validv
        ks = pl.ds(k_blk, 16)
        osc_v[ks] = jnp.where(ksel, jnp.full((16,), m_b), osc_v[ks])
        oidx_v[ks] = jnp.where(ksel, jnp.full((16,), i_b.astype(jnp.int32)),
                               oidx_v[ks])
        b0 = 4 * k
        b_blk = jnp.bitwise_and(b0, -16)
        p = jnp.bitwise_and(b0, 15)
        bsl = pl.ds(b_blk, 16)
        brow = obox_v[bsl]
        bv = jnp.where((iota_i == p) & validv, jnp.full((16,), x1_b),
             jnp.where((iota_i == p + 1) & validv, jnp.full((16,), y1_b),
             jnp.where((iota_i == p + 2) & validv, jnp.full((16,), x2_b),
             jnp.where((iota_i == p + 3) & validv, jnp.full((16,), y2_b),
                       brow))))
        obox_v[bsl] = bv

        karn = (x2_b - x1_b + 1.0) * (y2_b - y1_b + 1.0)
        nkx1 = jnp.where(valid, x1_b, -1.0e6)
        nky1 = jnp.where(valid, y1_b, -1.0e6)
        nkx2 = jnp.where(valid, x2_b, -1.0e6)
        nky2 = jnp.where(valid, y2_b, -1.0e6)
        nkar = jnp.where(valid, karn, 1.0)
        return (nkx1, nky1, nkx2, nky2, nkar)

    init = (jnp.float32(-1.0e6), jnp.float32(-1.0e6), jnp.float32(-1.0e6),
            jnp.float32(-1.0e6), jnp.float32(1.0))
    lax.fori_loop(0, K, one_round, init, unroll=False)

    @pl.when((cid == 0) & (wid == 0))
    def _():
        pltpu.sync_copy(osc_v, osc_h)
        pltpu.sync_copy(obox_v, obox_h)
        pltpu.sync_copy(oidx_v, oidx_h)


@jax.jit
def kernel(boxes, scores):
    f32 = jnp.float32
    pad = NPAD - N
    x1 = jnp.concatenate([boxes[:, 0], jnp.full((pad,), -1.0e6, f32)])
    y1 = jnp.concatenate([boxes[:, 1], jnp.full((pad,), -1.0e6, f32)])
    x2 = jnp.concatenate([boxes[:, 2], jnp.full((pad,), -1.0e6, f32)])
    y2 = jnp.concatenate([boxes[:, 3], jnp.full((pad,), -1.0e6, f32)])
    s = jnp.concatenate([scores.astype(f32), jnp.full((pad,), NEG, f32)])

    mesh = plsc.VectorSubcoreMesh(core_axis_name="c", subcore_axis_name="s",
                                  num_cores=1, num_subcores=NSUB)
    run = pl.kernel(
        _nms_body,
        out_type=(
            jax.ShapeDtypeStruct((KPAD,), f32),
            jax.ShapeDtypeStruct((4 * KPAD,), f32),
            jax.ShapeDtypeStruct((KPAD,), jnp.int32),
        ),
        mesh=mesh,
        compiler_params=pltpu.CompilerParams(needs_layout_passes=False),
        scratch_types=[
            pltpu.VMEM((SHARD,), f32),
            pltpu.VMEM((SHARD,), f32),
            pltpu.VMEM((SHARD,), f32),
            pltpu.VMEM((SHARD,), f32),
            pltpu.VMEM((SHARD,), f32),
            pltpu.VMEM((SHARD,), f32),
            pltpu.VMEM((16,), f32),
            pltpu.VMEM((NSUB * 16,), f32),
            pltpu.VMEM((KPAD,), f32),
            pltpu.VMEM((4 * KPAD,), f32),
            pltpu.VMEM((KPAD,), jnp.int32),
            pltpu.VMEM_SHARED((2 * NSUB * 16,), f32),
            pltpu.SemaphoreType.DMA,
        ],
    )
    osc, obox, oidx = run(x1, y1, x2, y2, s)
    return osc[:K], obox.reshape(KPAD, 4)[:K], oidx[:K]
